# baseline (device time: 104458 ns/iter reference)
import jax
import jax.numpy as jnp
from jax import lax
from jax.experimental import pallas as pl
from jax.experimental.pallas import tpu as pltpu

N_DEV = 32
PLANE = 8
NZ = 4


def kernel(x, w_mat):
    m, k_per = x.shape
    _, n = w_mat.shape
    m_per = m // N_DEV
    half = n // 2

    def body(x_ref, w_ref, out_ref, part_ref, rbuf1, rbuf2, sbuf,
             s1_send, s1_recv, s2_send, s2_recv):
        p = lax.axis_index("i")
        z = lax.div(p, PLANE)
        q = lax.rem(p, PLANE)

        def q_to_r(qq):
            yy = lax.div(qq, 2)
            xx = lax.rem(qq + yy, 2)
            return jnp.where(xx == 1, 1 + yy, lax.rem(8 - yy, 8))

        def r_to_q(rr):
            xx = jnp.where((rr >= 1) & (rr <= 4), 1, 0)
            yy = jnp.where(xx == 1, rr - 1, lax.rem(8 - rr, 8))
            return 2 * yy + lax.rem(xx + yy, 2)

        r = q_to_r(q)

        succ = z * PLANE + r_to_q(lax.rem(r + 1, 8))
        pred = z * PLANE + r_to_q(lax.rem(r + 7, 8))
        up = lax.rem(z + 1, NZ) * PLANE + q
        down = lax.rem(z + 3, NZ) * PLANE + q

        col0 = (0, half)
        sign = (-1, 1)
        dst1 = (succ, pred)
        dst2 = (up, down)

        def pref(b, dirn):
            return part_ref.at[pl.ds(b * m_per, m_per),
                               pl.ds(col0[dirn], half)]

        def pval(b, dirn):
            return part_ref[pl.ds(b * m_per, m_per),
                            pl.ds(col0[dirn], half)]

        rdmas = {}

        def zb_of(dirn, g):
            if dirn == 0:
                return lax.rem(z + 3 - g + NZ, NZ)
            return lax.rem(z + 1 + g, NZ)

        def p1_make(dirn, h, g):
            if h == 0:
                src = sbuf.at[dirn, g]
            else:
                src = rbuf1.at[dirn, h - 1, g]
            return pltpu.make_async_remote_copy(
                src_ref=src,
                dst_ref=rbuf1.at[dirn, h, g],
                send_sem=s1_send.at[dirn, h, g],
                recv_sem=s1_recv.at[dirn, h, g],
                device_id=(dst1[dirn],),
                device_id_type=pl.DeviceIdType.MESH,
            )

        def p2_make(dirn, h):
            if h == 0:
                src = rbuf1.at[dirn, PLANE - 2, 0]
            else:
                src = rbuf2.at[dirn, h - 1]
            return pltpu.make_async_remote_copy(
                src_ref=src,
                dst_ref=rbuf2.at[dirn, h],
                send_sem=s2_send.at[dirn, h],
                recv_sem=s2_recv.at[dirn, h],
                device_id=(dst2[dirn],),
                device_id_type=pl.DeviceIdType.MESH,
            )

        for dirn in (0, 1):
            for g in range(NZ):
                b0 = zb_of(dirn, g) * PLANE + r_to_q(
                    lax.rem(r + sign[dirn] + 16, 8)
                )
                sbuf[dirn, g] = jnp.dot(
                    x_ref[pl.ds(b0 * m_per, m_per), :],
                    w_ref[:, pl.ds(col0[dirn], half)],
                    preferred_element_type=jnp.float32,
                )

        barrier_sem = pltpu.get_barrier_semaphore()
        for nbr in (succ, pred, up, down):
            pl.semaphore_signal(
                barrier_sem, inc=1,
                device_id=(nbr,), device_id_type=pl.DeviceIdType.MESH,
            )
        pl.semaphore_wait(barrier_sem, 4)

        for g in range(NZ):
            for dirn in (0, 1):
                rd = p1_make(dirn, 0, g)
                rd.start()
                rdmas[(1, dirn, 0, g)] = rd

        part_ref[:, :] = jnp.dot(
            x_ref[:, :], w_ref[:, :], preferred_element_type=jnp.float32
        )

        for h in range(PLANE - 1):
            for g in range(NZ):
                for dirn in (0, 1):
                    rdmas[(1, dirn, h, g)].wait_recv()
                    rc = lax.rem(r + sign[dirn] * (2 + h) + 32, 8)
                    b = zb_of(dirn, g) * PLANE + r_to_q(rc)
                    rbuf1[dirn, h, g] = rbuf1[dirn, h, g] + pval(b, dirn)
                    if h < PLANE - 2:
                        rd = p1_make(dirn, h + 1, g)
                        rd.start()
                        rdmas[(1, dirn, h + 1, g)] = rd
                    else:
                        if g == 0:
                            rd = p2_make(dirn, 0)
                            rd.start()
                            rdmas[(2, dirn, 0, 0)] = rd
                        elif g < NZ - 1:
                            rdmas[(2, dirn, g - 1, 0)].wait_recv()
                            rbuf2[dirn, g - 1] = (
                                rbuf2[dirn, g - 1] + rbuf1[dirn, PLANE - 2, g]
                            )
                            rd = p2_make(dirn, g)
                            rd.start()
                            rdmas[(2, dirn, g, 0)] = rd
                        else:
                            rdmas[(2, dirn, g - 1, 0)].wait_recv()
                            y = (
                                rbuf2[dirn, g - 1]
                                + rbuf1[dirn, PLANE - 2, g]
                            )
                            yc = jnp.clip(y, -60.0, 60.0)
                            out_ref[:, pl.ds(col0[dirn], half)] = (
                                y / (1.0 + jnp.exp(-yc))
                            )

        for key in rdmas:
            rdmas[key].wait_send()

    return pl.pallas_call(
        body,
        out_shape=jax.ShapeDtypeStruct((m_per, n), jnp.float32),
        in_specs=[
            pl.BlockSpec(memory_space=pltpu.VMEM),
            pl.BlockSpec(memory_space=pltpu.VMEM),
        ],
        out_specs=pl.BlockSpec(memory_space=pltpu.VMEM),
        scratch_shapes=[
            pltpu.VMEM((m, n), jnp.float32),
            pltpu.VMEM((2, PLANE - 1, NZ, m_per, half), jnp.float32),
            pltpu.VMEM((2, NZ - 1, m_per, half), jnp.float32),
            pltpu.VMEM((2, NZ, m_per, half), jnp.float32),
            pltpu.SemaphoreType.DMA((2, PLANE - 1, NZ)),
            pltpu.SemaphoreType.DMA((2, PLANE - 1, NZ)),
            pltpu.SemaphoreType.DMA((2, NZ - 1)),
            pltpu.SemaphoreType.DMA((2, NZ - 1)),
        ],
        compiler_params=pltpu.CompilerParams(
            collective_id=0, vmem_limit_bytes=64 * 1024 * 1024
        ),
    )(x, w_mat)
